# Initial kernel scaffold; baseline (speedup 1.0000x reference)
import jax
import jax.numpy as jnp
from jax.experimental import pallas as pl


def kernel(x, edge_index, edge_attr, batch, W_l1, W_r1, W_e1, att1, b1, g1, be1, W_l2, W_r2, W_e2, att2, b2, g2, be2):
    raise NotImplementedError("write your pallas kernel here")


# speedup vs baseline: 11.7773x; 11.7773x over previous
import jax
import jax.numpy as jnp
from jax import lax
from jax.experimental import pallas as pl
from jax.experimental.pallas import tpu as pltpu
from jax.experimental.pallas import tpu_sc as plsc

N = 10000
E = 320000
B = 8
F_IN = 8
D1 = 256
C2 = 128

NP = 10112
E_PAD = 323584
K = 64
TBLK = 1264
STRIPE = NP // 16
DRP = 80

_mesh = plsc.VectorSubcoreMesh(core_axis_name="c", subcore_axis_name="s")
_f32 = jnp.float32
_i32 = jnp.int32


def _lanesum(v, lane):
    for sh in (1, 2, 4, 8):
        perm = lane ^ sh
        v = v + jnp.take_along_axis(v, perm, axis=0,
                                    mode="promise_in_bounds")
    return v


def _sc_common_init(zbuf, accum, den_acc, den_tile, nden, sid):
    for i in range(8):
        for v in range(8):
            zbuf[i, pl.ds(v * 16, 16)] = jnp.zeros((16,), _f32)

    def zd(i, _):
        for v in range(8):
            den_tile[i, pl.ds(v * 16, 16)] = jnp.zeros((16,), _f32)
        return 0

    lax.fori_loop(0, nden, zd, 0)
    r0 = sid * STRIPE

    def z1(j, _):
        pltpu.sync_copy(zbuf, accum.at[pl.ds(r0 + j * 8, 8)])
        return 0

    lax.fori_loop(0, STRIPE // 8, z1, 0)
    nchunk = nden // 8
    for t in range(2):
        j = sid + 16 * t

        @pl.when(j < nchunk)
        def _():
            pltpu.sync_copy(zbuf, den_acc.at[pl.ds(j * 8, 8)])


def _fill_iota(ref, nrows, lane, base):
    def f1(j, _):
        ref[pl.ds(j * 16, 16)] = lane + (base + j * 16)
        return 0
    lax.fori_loop(0, nrows // 16, f1, 0)


def _edge_loop(xl_ref, xr_ref, src, dst, attr, idx_s, idx_d, idx_dp, attr_v,
               xl_buf, xr_buf, val_buf, par_v, den_tile, accum,
               lane, base0, nchunks, heads2):
    wev = [par_v[pl.ds(v * 16, 16)] for v in range(8)]
    attv = [par_v[pl.ds(128 + v * 16, 16)] for v in range(8)]
    zero16 = jnp.zeros((16,), _f32)

    def chunk(j, _):
        base = base0 + j * K
        pltpu.sync_copy(src.at[pl.ds(base, K)], idx_s)
        pltpu.sync_copy(dst.at[pl.ds(base, K)], idx_d)
        pltpu.sync_copy(dst.at[pl.ds(base, K)], idx_dp.at[pl.ds(0, K)])
        pltpu.sync_copy(attr.at[pl.ds(base, K)], attr_v.at[pl.ds(0, K)])
        pltpu.sync_copy(xl_ref.at[idx_s], xl_buf)
        pltpu.sync_copy(xr_ref.at[idx_d], xr_buf)

        def edge(k, _):
            av = jnp.broadcast_to(attr_v[pl.ds(k, 16)][0], (16,))
            d16 = idx_dp[pl.ds(k, 16)]
            d0 = d16[0]
            xls = [xl_buf[k, pl.ds(v * 16, 16)] for v in range(8)]
            if heads2:
                acc0 = zero16
                acc1 = zero16
                for v in range(8):
                    z = xls[v] + xr_buf[k, pl.ds(v * 16, 16)] + av * wev[v]
                    z = jnp.where(z >= 0, z, 0.2 * z)
                    if v < 4:
                        acc0 = acc0 + attv[v] * z
                    else:
                        acc1 = acc1 + attv[v] * z
                ex0 = jnp.exp(_lanesum(acc0, lane))
                ex1 = jnp.exp(_lanesum(acc1, lane))
                for v in range(4):
                    val_buf[k, pl.ds(v * 16, 16)] = ex0 * xls[v]
                for v in range(4, 8):
                    val_buf[k, pl.ds(v * 16, 16)] = ex1 * xls[v]
                cbase = (d0 & 63) * 2
                row = lax.shift_right_logical(d0, 6)
                cb = cbase & 0x70
                liv = jnp.broadcast_to(cbase & 15, (16,))
                cur = den_tile[row, pl.ds(cb, 16)]
                cur = (cur + jnp.where(lane == liv, ex0, zero16)
                       + jnp.where(lane == liv + 1, ex1, zero16))
                den_tile[row, pl.ds(cb, 16)] = cur
            else:
                acc = zero16
                for v in range(8):
                    z = xls[v] + xr_buf[k, pl.ds(v * 16, 16)] + av * wev[v]
                    z = jnp.where(z >= 0, z, 0.2 * z)
                    acc = acc + attv[v] * z
                ex = jnp.exp(_lanesum(acc, lane))
                for v in range(8):
                    val_buf[k, pl.ds(v * 16, 16)] = ex * xls[v]
                c = d0 & 127
                row = lax.shift_right_logical(d0, 7)
                cb = c & 0x70
                liv = jnp.broadcast_to(c & 15, (16,))
                cur = den_tile[row, pl.ds(cb, 16)]
                den_tile[row, pl.ds(cb, 16)] = (
                    cur + jnp.where(lane == liv, ex, zero16))
            return 0

        lax.fori_loop(0, K, edge, 0)
        pltpu.sync_copy(val_buf, accum.at[idx_d], add=True)
        return 0

    lax.fori_loop(0, nchunks, chunk, 0)


def _merge_den(den_tile, den_acc, idx_mA, idx_mB, heads2):
    pltpu.sync_copy(den_tile.at[pl.ds(0, DRP)], den_acc.at[idx_mA], add=True)
    if heads2:
        pltpu.sync_copy(den_tile.at[pl.ds(DRP, DRP)], den_acc.at[idx_mB],
                        add=True)


def _copy_out(accum, den_acc, out_num, out_den, nden, sid):
    r0 = sid * STRIPE
    pltpu.sync_copy(accum.at[pl.ds(r0, STRIPE)], out_num.at[pl.ds(r0, STRIPE)])
    nchunk = nden // 8
    for t in range(2):
        j = sid + 16 * t

        @pl.when(j < nchunk)
        def _():
            pltpu.sync_copy(den_acc.at[pl.ds(j * 8, 8)],
                            out_den.at[pl.ds(j * 8, 8)])


def _gat1_body(xl_a, xr_a, xl_b, xr_b, src, dst, attr, par_a, par_b,
               na, nb, da, db,
               idx_s, idx_d, idx_dp, attr_v, xl_buf, xr_buf, val_buf, par_v,
               zbuf, idx_mA, idx_mB, den_tile, accum, den_acc):
    cid = lax.axis_index("c")
    sid = lax.axis_index("s")
    lane = lax.iota(_i32, 16)
    nden = 2 * DRP
    _sc_common_init(zbuf, accum, den_acc, den_tile, nden, sid)
    _fill_iota(idx_mA, DRP, lane, 0)
    _fill_iota(idx_mB, DRP, lane, DRP)
    plsc.subcore_barrier()

    ept = E_PAD // 16

    @pl.when(cid == 0)
    def _():
        pltpu.sync_copy(par_a, par_v)
        _edge_loop(xl_a, xr_a, src, dst, attr, idx_s, idx_d, idx_dp, attr_v,
                   xl_buf, xr_buf, val_buf, par_v, den_tile, accum,
                   lane, sid * ept, ept // K, True)

    @pl.when(cid == 1)
    def _():
        pltpu.sync_copy(par_b, par_v)
        _edge_loop(xl_b, xr_b, src, dst, attr, idx_s, idx_d, idx_dp, attr_v,
                   xl_buf, xr_buf, val_buf, par_v, den_tile, accum,
                   lane, sid * ept, ept // K, True)

    _merge_den(den_tile, den_acc, idx_mA, idx_mB, True)
    plsc.subcore_barrier()

    @pl.when(cid == 0)
    def _():
        _copy_out(accum, den_acc, na, da, nden, sid)

    @pl.when(cid == 1)
    def _():
        _copy_out(accum, den_acc, nb, db, nden, sid)


def _gat2_body(xl, xr, src, dst, attr, par,
               na, nb, da, db,
               idx_s, idx_d, idx_dp, attr_v, xl_buf, xr_buf, val_buf, par_v,
               zbuf, idx_mA, idx_mB, den_tile, accum, den_acc):
    cid = lax.axis_index("c")
    sid = lax.axis_index("s")
    lane = lax.iota(_i32, 16)
    _sc_common_init(zbuf, accum, den_acc, den_tile, DRP, sid)
    _fill_iota(idx_mA, DRP, lane, 0)
    plsc.subcore_barrier()

    pltpu.sync_copy(par, par_v)
    wid = cid * 16 + sid
    ept = E_PAD // 32
    _edge_loop(xl, xr, src, dst, attr, idx_s, idx_d, idx_dp, attr_v,
               xl_buf, xr_buf, val_buf, par_v, den_tile, accum,
               lane, wid * ept, ept // K, False)
    _merge_den(den_tile, den_acc, idx_mA, idx_mB, False)
    plsc.subcore_barrier()

    @pl.when(cid == 0)
    def _():
        _copy_out(accum, den_acc, na, da, DRP, sid)

    @pl.when(cid == 1)
    def _():
        _copy_out(accum, den_acc, nb, db, DRP, sid)


def _sc_scratch(nden):
    return [
        pltpu.VMEM((K,), _i32),
        pltpu.VMEM((K,), _i32),
        pltpu.VMEM((K + 16,), _i32),
        pltpu.VMEM((K + 16,), _f32),
        pltpu.VMEM((K, 128), _f32),
        pltpu.VMEM((K, 128), _f32),
        pltpu.VMEM((K, 128), _f32),
        pltpu.VMEM((D1,), _f32),
        pltpu.VMEM((8, 128), _f32),
        pltpu.VMEM((DRP,), _i32),
        pltpu.VMEM((DRP,), _i32),
        pltpu.VMEM((nden, 128), _f32),
        pltpu.VMEM_SHARED((NP, 128), _f32),
        pltpu.VMEM_SHARED((nden, 128), _f32),
    ]


@jax.jit
def _gat1_call(xl_a, xr_a, xl_b, xr_b, src, dst, attr, par_a, par_b):
    return pl.kernel(
        _gat1_body,
        out_type=[jax.ShapeDtypeStruct((NP, 128), _f32)] * 2
        + [jax.ShapeDtypeStruct((2 * DRP, 128), _f32)] * 2,
        mesh=_mesh,
        scratch_types=_sc_scratch(2 * DRP),
    )(xl_a, xr_a, xl_b, xr_b, src, dst, attr, par_a, par_b)


@jax.jit
def _gat2_call(xl, xr, src, dst, attr, par):
    return pl.kernel(
        _gat2_body,
        out_type=[jax.ShapeDtypeStruct((NP, 128), _f32)] * 2
        + [jax.ShapeDtypeStruct((DRP, 128), _f32)] * 2,
        mesh=_mesh,
        scratch_types=_sc_scratch(DRP),
    )(xl, xr, src, dst, attr, par)


def _mm1_body(x_ref, wl_ref, wr_ref, o1, o2, o3, o4):
    xb = x_ref[...]
    xl = jnp.dot(xb, wl_ref[...], preferred_element_type=_f32)
    xr = jnp.dot(xb, wr_ref[...], preferred_element_type=_f32)
    o1[...] = xl[:, :128]
    o2[...] = xl[:, 128:]
    o3[...] = xr[:, :128]
    o4[...] = xr[:, 128:]


def _stage1(xp, wl, wr):
    return pl.pallas_call(
        _mm1_body,
        grid=(NP // TBLK,),
        in_specs=[
            pl.BlockSpec((TBLK, F_IN), lambda i: (i, 0)),
            pl.BlockSpec((F_IN, D1), lambda i: (0, 0)),
            pl.BlockSpec((F_IN, D1), lambda i: (0, 0)),
        ],
        out_specs=[pl.BlockSpec((TBLK, 128), lambda i: (i, 0))] * 4,
        out_shape=[jax.ShapeDtypeStruct((NP, 128), _f32)] * 4,
    )(xp, wl, wr)


def _ln_elu(h, g, b):
    mu = jnp.mean(h, axis=1, keepdims=True)
    var = jnp.mean((h - mu) ** 2, axis=1, keepdims=True)
    hn = (h - mu) * lax.rsqrt(var + 1e-5) * g + b
    return jnp.where(hn > 0, hn, jnp.exp(hn) - 1.0)


def _upd1_body(na_ref, nb_ref, d0_ref, d1_ref, d2_ref, d3_ref,
               b1_ref, g1_ref, be1_ref, wl2_ref, wr2_ref,
               xl2_ref, xr2_ref):
    num = jnp.concatenate([na_ref[...], nb_ref[...]], axis=1)
    dens = [d0_ref[0, 0, :], d1_ref[0, 0, :], d2_ref[0, 0, :], d3_ref[0, 0, :]]
    den = jnp.concatenate(
        [jnp.broadcast_to(d[:, None], (TBLK, 64)) for d in dens], axis=1)
    h = num / (den + 1e-16) + b1_ref[...]
    h = _ln_elu(h, g1_ref[...], be1_ref[...])
    xl2_ref[...] = jnp.dot(h, wl2_ref[...], preferred_element_type=_f32)
    xr2_ref[...] = jnp.dot(h, wr2_ref[...], preferred_element_type=_f32)


def _stage3(na, nb, d0, d1, d2, d3, b1, g1, be1, wl2, wr2):
    dspec = pl.BlockSpec((1, 1, TBLK), lambda i: (i, 0, 0))
    pspec = pl.BlockSpec((1, D1), lambda i: (0, 0))
    return pl.pallas_call(
        _upd1_body,
        grid=(NP // TBLK,),
        in_specs=[
            pl.BlockSpec((TBLK, 128), lambda i: (i, 0)),
            pl.BlockSpec((TBLK, 128), lambda i: (i, 0)),
            dspec, dspec, dspec, dspec,
            pspec, pspec, pspec,
            pl.BlockSpec((D1, C2), lambda i: (0, 0)),
            pl.BlockSpec((D1, C2), lambda i: (0, 0)),
        ],
        out_specs=[pl.BlockSpec((TBLK, C2), lambda i: (i, 0))] * 2,
        out_shape=[jax.ShapeDtypeStruct((NP, C2), _f32)] * 2,
    )(na, nb, d0, d1, d2, d3, b1, g1, be1, wl2, wr2)


def _pool_body(na_ref, nb_ref, da_ref, db_ref, b2_ref, g2_ref, be2_ref,
               bt_ref, out_ref, acc, cnt):
    i = pl.program_id(0)
    num = na_ref[...] + nb_ref[...]
    den = (da_ref[0, 0, :] + db_ref[0, 0, :])[:, None]
    h = num / (den + 1e-16) + b2_ref[...]
    h = _ln_elu(h, g2_ref[...], be2_ref[...])
    bt = bt_ref[0, 0, :]
    onehot = (bt[None, :] ==
              lax.broadcasted_iota(_i32, (B, TBLK), 0)).astype(_f32)
    pm = jnp.dot(onehot, h, preferred_element_type=_f32)
    rs = jnp.broadcast_to(jnp.sum(onehot, axis=1, keepdims=True), (B, C2))

    @pl.when(i == 0)
    def _():
        acc[...] = pm
        cnt[...] = rs

    @pl.when(i > 0)
    def _():
        acc[...] += pm
        cnt[...] += rs

    @pl.when(i == NP // TBLK - 1)
    def _():
        out_ref[...] = acc[...] / jnp.maximum(cnt[...], 1.0)


def _stage5(na, nb, da, db, b2, g2, be2, bt):
    dspec = pl.BlockSpec((1, 1, TBLK), lambda i: (i, 0, 0))
    pspec = pl.BlockSpec((1, C2), lambda i: (0, 0))
    return pl.pallas_call(
        _pool_body,
        grid=(NP // TBLK,),
        in_specs=[
            pl.BlockSpec((TBLK, 128), lambda i: (i, 0)),
            pl.BlockSpec((TBLK, 128), lambda i: (i, 0)),
            dspec, dspec,
            pspec, pspec, pspec,
            dspec,
        ],
        out_specs=pl.BlockSpec((B, C2), lambda i: (0, 0)),
        out_shape=jax.ShapeDtypeStruct((B, C2), _f32),
        scratch_shapes=[pltpu.VMEM((B, C2), _f32), pltpu.VMEM((B, C2), _f32)],
    )(na, nb, da, db, b2, g2, be2, bt)


def _to_blocks(v):
    return v.reshape(NP // TBLK, 1, TBLK)


def kernel(x, edge_index, edge_attr, batch, W_l1, W_r1, W_e1, att1, b1, g1,
           be1, W_l2, W_r2, W_e2, att2, b2, g2, be2):
    x = x.astype(_f32)
    xp = jnp.zeros((NP, F_IN), _f32).at[:N].set(x)
    src = edge_index[0].astype(_i32)
    dst = edge_index[1].astype(_i32)
    pad_e = E_PAD - E
    srcp = jnp.concatenate([src, jnp.full((pad_e,), N, _i32)])
    dstp = jnp.concatenate([dst, jnp.full((pad_e,), N, _i32)])
    attrp = jnp.concatenate([edge_attr[:, 0].astype(_f32),
                             jnp.zeros((pad_e,), _f32)])
    btp = _to_blocks(jnp.concatenate([batch.astype(_i32),
                                      jnp.full((NP - N,), B, _i32)]))

    par_a1 = jnp.concatenate([W_e1[0, :128], att1[0:2].reshape(128)])
    par_b1 = jnp.concatenate([W_e1[0, 128:], att1[2:4].reshape(128)])
    par2 = jnp.concatenate([W_e2[0], att2[0]])

    xl_a, xl_b, xr_a, xr_b = _stage1(xp, W_l1, W_r1)
    na1, nb1, da1, db1 = _gat1_call(xl_a, xr_a, xl_b, xr_b, srcp, dstp,
                                    attrp, par_a1, par_b1)
    da1f = da1.reshape(DRP * 128, 2)
    db1f = db1.reshape(DRP * 128, 2)
    d10 = _to_blocks(da1f[:NP, 0])
    d11 = _to_blocks(da1f[:NP, 1])
    d12 = _to_blocks(db1f[:NP, 0])
    d13 = _to_blocks(db1f[:NP, 1])
    xl2, xr2 = _stage3(na1, nb1, d10, d11, d12, d13,
                       b1.reshape(1, D1), g1.reshape(1, D1),
                       be1.reshape(1, D1), W_l2, W_r2)
    na2, nb2, da2, db2 = _gat2_call(xl2, xr2, srcp, dstp, attrp, par2)
    d2a = _to_blocks(da2.reshape(-1)[:NP])
    d2b = _to_blocks(db2.reshape(-1)[:NP])
    return _stage5(na2, nb2, d2a, d2b, b2.reshape(1, C2), g2.reshape(1, C2),
                   be2.reshape(1, C2), btp)


# speedup vs baseline: 16.3001x; 1.3840x over previous
import jax
import jax.numpy as jnp
from jax import lax
from jax.experimental import pallas as pl
from jax.experimental.pallas import tpu as pltpu
from jax.experimental.pallas import tpu_sc as plsc

N = 10000
E = 320000
B = 8
F_IN = 8
D1 = 256
C2 = 128

NP = 10112
E_PAD = 323584
K = 64
TBLK = 1264
STRIPE = NP // 16
DRP = 80

_mesh = plsc.VectorSubcoreMesh(core_axis_name="c", subcore_axis_name="s")
_f32 = jnp.float32
_i32 = jnp.int32


def _lanesum(v, lane):
    for sh in (1, 2, 4, 8):
        perm = lane ^ sh
        v = v + jnp.take_along_axis(v, perm, axis=0,
                                    mode="promise_in_bounds")
    return v


def _sc_common_init(zbuf, accum, den_acc, den_tile, nden, sid):
    for i in range(8):
        for v in range(8):
            zbuf[i, pl.ds(v * 16, 16)] = jnp.zeros((16,), _f32)

    def zd(i, _):
        for v in range(8):
            den_tile[i, pl.ds(v * 16, 16)] = jnp.zeros((16,), _f32)
        return 0

    lax.fori_loop(0, nden, zd, 0)
    r0 = sid * STRIPE

    def z1(j, _):
        pltpu.sync_copy(zbuf, accum.at[pl.ds(r0 + j * 8, 8)])
        return 0

    lax.fori_loop(0, STRIPE // 8, z1, 0)
    nchunk = nden // 8
    for t in range(2):
        j = sid + 16 * t

        @pl.when(j < nchunk)
        def _():
            pltpu.sync_copy(zbuf, den_acc.at[pl.ds(j * 8, 8)])


def _fill_iota(ref, nrows, lane, base):
    def f1(j, _):
        ref[pl.ds(j * 16, 16)] = lane + (base + j * 16)
        return 0
    lax.fori_loop(0, nrows // 16, f1, 0)


def _edge_loop(xl_ref, xr_ref, src, dst, attr, idx_s, idx_d, idx_dp, attr_v,
               xl_buf, xr_buf, val_buf, par_v, den_tile, accum,
               semA, semB, lane, base0, nchunks, heads2):
    wev = [par_v[pl.ds(v * 16, 16)] for v in range(8)]
    attv = [par_v[pl.ds(128 + v * 16, 16)] for v in range(8)]
    zero16 = jnp.zeros((16,), _f32)

    def chunk(j, _):
        base = base0 + j * K
        c1 = pltpu.async_copy(src.at[pl.ds(base, K)], idx_s, semA)
        c2 = pltpu.async_copy(dst.at[pl.ds(base, K)], idx_d, semA)
        c3 = pltpu.async_copy(attr.at[pl.ds(base, K)], attr_v.at[pl.ds(0, K)],
                              semA)
        c1.wait()
        c2.wait()
        c3.wait()
        for t in range(K // 16):
            idx_dp[pl.ds(t * 16, 16)] = idx_d[pl.ds(t * 16, 16)]
        g1 = pltpu.async_copy(xl_ref.at[idx_s], xl_buf, semB)
        g2 = pltpu.async_copy(xr_ref.at[idx_d], xr_buf, semB)
        g1.wait()
        g2.wait()

        def edge(k, _):
            av = jnp.broadcast_to(attr_v[pl.ds(k, 16)][0], (16,))
            d16 = idx_dp[pl.ds(k, 16)]
            d0 = d16[0]
            xls = [xl_buf[k, pl.ds(v * 16, 16)] for v in range(8)]
            if heads2:
                acc0 = zero16
                acc1 = zero16
                for v in range(8):
                    z = xls[v] + xr_buf[k, pl.ds(v * 16, 16)] + av * wev[v]
                    z = jnp.where(z >= 0, z, 0.2 * z)
                    if v < 4:
                        acc0 = acc0 + attv[v] * z
                    else:
                        acc1 = acc1 + attv[v] * z
                ex0 = jnp.exp(_lanesum(acc0, lane))
                ex1 = jnp.exp(_lanesum(acc1, lane))
                for v in range(4):
                    val_buf[k, pl.ds(v * 16, 16)] = ex0 * xls[v]
                for v in range(4, 8):
                    val_buf[k, pl.ds(v * 16, 16)] = ex1 * xls[v]
                cbase = (d0 & 63) * 2
                row = lax.shift_right_logical(d0, 6)
                cb = cbase & 0x70
                liv = jnp.broadcast_to(cbase & 15, (16,))
                cur = den_tile[row, pl.ds(cb, 16)]
                cur = (cur + jnp.where(lane == liv, ex0, zero16)
                       + jnp.where(lane == liv + 1, ex1, zero16))
                den_tile[row, pl.ds(cb, 16)] = cur
            else:
                acc = zero16
                for v in range(8):
                    z = xls[v] + xr_buf[k, pl.ds(v * 16, 16)] + av * wev[v]
                    z = jnp.where(z >= 0, z, 0.2 * z)
                    acc = acc + attv[v] * z
                ex = jnp.exp(_lanesum(acc, lane))
                for v in range(8):
                    val_buf[k, pl.ds(v * 16, 16)] = ex * xls[v]
                c = d0 & 127
                row = lax.shift_right_logical(d0, 7)
                cb = c & 0x70
                liv = jnp.broadcast_to(c & 15, (16,))
                cur = den_tile[row, pl.ds(cb, 16)]
                den_tile[row, pl.ds(cb, 16)] = (
                    cur + jnp.where(lane == liv, ex, zero16))
            return 0

        lax.fori_loop(0, K, edge, 0)
        pltpu.sync_copy(val_buf, accum.at[idx_d], add=True)
        return 0

    lax.fori_loop(0, nchunks, chunk, 0)


def _merge_den(den_tile, den_acc, idx_mA, idx_mB, heads2):
    pltpu.sync_copy(den_tile.at[pl.ds(0, DRP)], den_acc.at[idx_mA], add=True)
    if heads2:
        pltpu.sync_copy(den_tile.at[pl.ds(DRP, DRP)], den_acc.at[idx_mB],
                        add=True)


def _copy_out(accum, den_acc, out_num, out_den, nden, sid):
    r0 = sid * STRIPE
    pltpu.sync_copy(accum.at[pl.ds(r0, STRIPE)], out_num.at[pl.ds(r0, STRIPE)])
    nchunk = nden // 8
    for t in range(2):
        j = sid + 16 * t

        @pl.when(j < nchunk)
        def _():
            pltpu.sync_copy(den_acc.at[pl.ds(j * 8, 8)],
                            out_den.at[pl.ds(j * 8, 8)])


def _gat1_body(xl_a, xr_a, xl_b, xr_b, src, dst, attr, par_a, par_b,
               na, nb, da, db,
               idx_s, idx_d, idx_dp, attr_v, xl_buf, xr_buf, val_buf, par_v,
               zbuf, idx_mA, idx_mB, den_tile, accum, den_acc, semA, semB):
    cid = lax.axis_index("c")
    sid = lax.axis_index("s")
    lane = lax.iota(_i32, 16)
    nden = 2 * DRP
    _sc_common_init(zbuf, accum, den_acc, den_tile, nden, sid)
    _fill_iota(idx_mA, DRP, lane, 0)
    _fill_iota(idx_mB, DRP, lane, DRP)
    plsc.subcore_barrier()

    ept = E_PAD // 16

    @pl.when(cid == 0)
    def _():
        pltpu.sync_copy(par_a, par_v)
        _edge_loop(xl_a, xr_a, src, dst, attr, idx_s, idx_d, idx_dp, attr_v,
                   xl_buf, xr_buf, val_buf, par_v, den_tile, accum,
                   semA, semB, lane, sid * ept, ept // K, True)

    @pl.when(cid == 1)
    def _():
        pltpu.sync_copy(par_b, par_v)
        _edge_loop(xl_b, xr_b, src, dst, attr, idx_s, idx_d, idx_dp, attr_v,
                   xl_buf, xr_buf, val_buf, par_v, den_tile, accum,
                   semA, semB, lane, sid * ept, ept // K, True)

    _merge_den(den_tile, den_acc, idx_mA, idx_mB, True)
    plsc.subcore_barrier()

    @pl.when(cid == 0)
    def _():
        _copy_out(accum, den_acc, na, da, nden, sid)

    @pl.when(cid == 1)
    def _():
        _copy_out(accum, den_acc, nb, db, nden, sid)


def _gat2_body(xl, xr, src, dst, attr, par,
               na, nb, da, db,
               idx_s, idx_d, idx_dp, attr_v, xl_buf, xr_buf, val_buf, par_v,
               zbuf, idx_mA, idx_mB, den_tile, accum, den_acc, semA, semB):
    cid = lax.axis_index("c")
    sid = lax.axis_index("s")
    lane = lax.iota(_i32, 16)
    _sc_common_init(zbuf, accum, den_acc, den_tile, DRP, sid)
    _fill_iota(idx_mA, DRP, lane, 0)
    plsc.subcore_barrier()

    pltpu.sync_copy(par, par_v)
    wid = cid * 16 + sid
    ept = E_PAD // 32
    _edge_loop(xl, xr, src, dst, attr, idx_s, idx_d, idx_dp, attr_v,
               xl_buf, xr_buf, val_buf, par_v, den_tile, accum,
               semA, semB, lane, wid * ept, ept // K, False)
    _merge_den(den_tile, den_acc, idx_mA, idx_mB, False)
    plsc.subcore_barrier()

    @pl.when(cid == 0)
    def _():
        _copy_out(accum, den_acc, na, da, DRP, sid)

    @pl.when(cid == 1)
    def _():
        _copy_out(accum, den_acc, nb, db, DRP, sid)


def _sc_scratch(nden):
    return [
        pltpu.VMEM((K,), _i32),
        pltpu.VMEM((K,), _i32),
        pltpu.VMEM((K + 16,), _i32),
        pltpu.VMEM((K + 16,), _f32),
        pltpu.VMEM((K, 128), _f32),
        pltpu.VMEM((K, 128), _f32),
        pltpu.VMEM((K, 128), _f32),
        pltpu.VMEM((D1,), _f32),
        pltpu.VMEM((8, 128), _f32),
        pltpu.VMEM((DRP,), _i32),
        pltpu.VMEM((DRP,), _i32),
        pltpu.VMEM((nden, 128), _f32),
        pltpu.VMEM_SHARED((NP, 128), _f32),
        pltpu.VMEM_SHARED((nden, 128), _f32),
        pltpu.SemaphoreType.DMA,
        pltpu.SemaphoreType.DMA,
    ]


@jax.jit
def _gat1_call(xl_a, xr_a, xl_b, xr_b, src, dst, attr, par_a, par_b):
    return pl.kernel(
        _gat1_body,
        out_type=[jax.ShapeDtypeStruct((NP, 128), _f32)] * 2
        + [jax.ShapeDtypeStruct((2 * DRP, 128), _f32)] * 2,
        mesh=_mesh,
        scratch_types=_sc_scratch(2 * DRP),
    )(xl_a, xr_a, xl_b, xr_b, src, dst, attr, par_a, par_b)


@jax.jit
def _gat2_call(xl, xr, src, dst, attr, par):
    return pl.kernel(
        _gat2_body,
        out_type=[jax.ShapeDtypeStruct((NP, 128), _f32)] * 2
        + [jax.ShapeDtypeStruct((DRP, 128), _f32)] * 2,
        mesh=_mesh,
        scratch_types=_sc_scratch(DRP),
    )(xl, xr, src, dst, attr, par)


def _mm1_body(x_ref, wl_ref, wr_ref, o1, o2, o3, o4):
    xb = x_ref[...]
    xl = jnp.dot(xb, wl_ref[...], preferred_element_type=_f32)
    xr = jnp.dot(xb, wr_ref[...], preferred_element_type=_f32)
    o1[...] = xl[:, :128]
    o2[...] = xl[:, 128:]
    o3[...] = xr[:, :128]
    o4[...] = xr[:, 128:]


def _stage1(xp, wl, wr):
    return pl.pallas_call(
        _mm1_body,
        grid=(NP // TBLK,),
        in_specs=[
            pl.BlockSpec((TBLK, F_IN), lambda i: (i, 0)),
            pl.BlockSpec((F_IN, D1), lambda i: (0, 0)),
            pl.BlockSpec((F_IN, D1), lambda i: (0, 0)),
        ],
        out_specs=[pl.BlockSpec((TBLK, 128), lambda i: (i, 0))] * 4,
        out_shape=[jax.ShapeDtypeStruct((NP, 128), _f32)] * 4,
    )(xp, wl, wr)


def _ln_elu(h, g, b):
    mu = jnp.mean(h, axis=1, keepdims=True)
    var = jnp.mean((h - mu) ** 2, axis=1, keepdims=True)
    hn = (h - mu) * lax.rsqrt(var + 1e-5) * g + b
    return jnp.where(hn > 0, hn, jnp.exp(hn) - 1.0)


def _upd1_body(na_ref, nb_ref, d0_ref, d1_ref, d2_ref, d3_ref,
               b1_ref, g1_ref, be1_ref, wl2_ref, wr2_ref,
               xl2_ref, xr2_ref):
    num = jnp.concatenate([na_ref[...], nb_ref[...]], axis=1)
    dens = [d0_ref[0, 0, :], d1_ref[0, 0, :], d2_ref[0, 0, :], d3_ref[0, 0, :]]
    den = jnp.concatenate(
        [jnp.broadcast_to(d[:, None], (TBLK, 64)) for d in dens], axis=1)
    h = num / (den + 1e-16) + b1_ref[...]
    h = _ln_elu(h, g1_ref[...], be1_ref[...])
    xl2_ref[...] = jnp.dot(h, wl2_ref[...], preferred_element_type=_f32)
    xr2_ref[...] = jnp.dot(h, wr2_ref[...], preferred_element_type=_f32)


def _stage3(na, nb, d0, d1, d2, d3, b1, g1, be1, wl2, wr2):
    dspec = pl.BlockSpec((1, 1, TBLK), lambda i: (i, 0, 0))
    pspec = pl.BlockSpec((1, D1), lambda i: (0, 0))
    return pl.pallas_call(
        _upd1_body,
        grid=(NP // TBLK,),
        in_specs=[
            pl.BlockSpec((TBLK, 128), lambda i: (i, 0)),
            pl.BlockSpec((TBLK, 128), lambda i: (i, 0)),
            dspec, dspec, dspec, dspec,
            pspec, pspec, pspec,
            pl.BlockSpec((D1, C2), lambda i: (0, 0)),
            pl.BlockSpec((D1, C2), lambda i: (0, 0)),
        ],
        out_specs=[pl.BlockSpec((TBLK, C2), lambda i: (i, 0))] * 2,
        out_shape=[jax.ShapeDtypeStruct((NP, C2), _f32)] * 2,
    )(na, nb, d0, d1, d2, d3, b1, g1, be1, wl2, wr2)


def _pool_body(na_ref, nb_ref, da_ref, db_ref, b2_ref, g2_ref, be2_ref,
               bt_ref, out_ref, acc, cnt):
    i = pl.program_id(0)
    num = na_ref[...] + nb_ref[...]
    den = (da_ref[0, 0, :] + db_ref[0, 0, :])[:, None]
    h = num / (den + 1e-16) + b2_ref[...]
    h = _ln_elu(h, g2_ref[...], be2_ref[...])
    bt = bt_ref[0, 0, :]
    onehot = (bt[None, :] ==
              lax.broadcasted_iota(_i32, (B, TBLK), 0)).astype(_f32)
    pm = jnp.dot(onehot, h, preferred_element_type=_f32)
    rs = jnp.broadcast_to(jnp.sum(onehot, axis=1, keepdims=True), (B, C2))

    @pl.when(i == 0)
    def _():
        acc[...] = pm
        cnt[...] = rs

    @pl.when(i > 0)
    def _():
        acc[...] += pm
        cnt[...] += rs

    @pl.when(i == NP // TBLK - 1)
    def _():
        out_ref[...] = acc[...] / jnp.maximum(cnt[...], 1.0)


def _stage5(na, nb, da, db, b2, g2, be2, bt):
    dspec = pl.BlockSpec((1, 1, TBLK), lambda i: (i, 0, 0))
    pspec = pl.BlockSpec((1, C2), lambda i: (0, 0))
    return pl.pallas_call(
        _pool_body,
        grid=(NP // TBLK,),
        in_specs=[
            pl.BlockSpec((TBLK, 128), lambda i: (i, 0)),
            pl.BlockSpec((TBLK, 128), lambda i: (i, 0)),
            dspec, dspec,
            pspec, pspec, pspec,
            dspec,
        ],
        out_specs=pl.BlockSpec((B, C2), lambda i: (0, 0)),
        out_shape=jax.ShapeDtypeStruct((B, C2), _f32),
        scratch_shapes=[pltpu.VMEM((B, C2), _f32), pltpu.VMEM((B, C2), _f32)],
    )(na, nb, da, db, b2, g2, be2, bt)


def _to_blocks(v):
    return v.reshape(NP // TBLK, 1, TBLK)


def kernel(x, edge_index, edge_attr, batch, W_l1, W_r1, W_e1, att1, b1, g1,
           be1, W_l2, W_r2, W_e2, att2, b2, g2, be2):
    x = x.astype(_f32)
    xp = jnp.zeros((NP, F_IN), _f32).at[:N].set(x)
    src = edge_index[0].astype(_i32)
    dst = edge_index[1].astype(_i32)
    pad_e = E_PAD - E
    srcp = jnp.concatenate([src, jnp.full((pad_e,), N, _i32)])
    dstp = jnp.concatenate([dst, jnp.full((pad_e,), N, _i32)])
    attrp = jnp.concatenate([edge_attr[:, 0].astype(_f32),
                             jnp.zeros((pad_e,), _f32)])
    btp = _to_blocks(jnp.concatenate([batch.astype(_i32),
                                      jnp.full((NP - N,), B, _i32)]))

    par_a1 = jnp.concatenate([W_e1[0, :128], att1[0:2].reshape(128)])
    par_b1 = jnp.concatenate([W_e1[0, 128:], att1[2:4].reshape(128)])
    par2 = jnp.concatenate([W_e2[0], att2[0]])

    xl_a, xl_b, xr_a, xr_b = _stage1(xp, W_l1, W_r1)
    na1, nb1, da1, db1 = _gat1_call(xl_a, xr_a, xl_b, xr_b, srcp, dstp,
                                    attrp, par_a1, par_b1)
    da1f = da1.reshape(DRP * 128, 2)
    db1f = db1.reshape(DRP * 128, 2)
    d10 = _to_blocks(da1f[:NP, 0])
    d11 = _to_blocks(da1f[:NP, 1])
    d12 = _to_blocks(db1f[:NP, 0])
    d13 = _to_blocks(db1f[:NP, 1])
    xl2, xr2 = _stage3(na1, nb1, d10, d11, d12, d13,
                       b1.reshape(1, D1), g1.reshape(1, D1),
                       be1.reshape(1, D1), W_l2, W_r2)
    na2, nb2, da2, db2 = _gat2_call(xl2, xr2, srcp, dstp, attrp, par2)
    d2a = _to_blocks(da2.reshape(-1)[:NP])
    d2b = _to_blocks(db2.reshape(-1)[:NP])
    return _stage5(na2, nb2, d2a, d2b, b2.reshape(1, C2), g2.reshape(1, C2),
                   be2.reshape(1, C2), btp)
